# baseline (speedup 1.0000x reference)
import functools

import jax
import jax.numpy as jnp
from jax import lax
from jax.experimental import pallas as pl
from jax.experimental.pallas import tpu as pltpu
from jax.experimental.pallas import tpu_sc as plsc

N_ATOMS = 100000
N_TYPES = 50
CHUNK = 768
CHUNK2 = 512
SC_COLS = 32 * (CHUNK + CHUNK2)
GROUPS = CHUNK // 16
GROUPS2 = CHUNK2 // 16
NW = 32

TC_BLOCK = 20480
TC_FIRST = SC_COLS // TC_BLOCK
N_TC_BLOCKS = pl.cdiv(N_ATOMS, TC_BLOCK) - TC_FIRST

DUP_BLOCK = 50176
N_DUP_BLOCKS = pl.cdiv(N_ATOMS, DUP_BLOCK)


@functools.partial(
    pl.kernel,
    out_type=jax.ShapeDtypeStruct((N_TYPES, N_ATOMS), jnp.float32),
    mesh=plsc.VectorSubcoreMesh(core_axis_name="c", subcore_axis_name="s"),
    scratch_types=[
        pltpu.VMEM((CHUNK,), jnp.int32),
        pltpu.VMEM((N_TYPES, CHUNK), jnp.float32),
    ],
    compiler_params=pltpu.CompilerParams(
        needs_layout_passes=False, skip_device_barrier=True
    ),
)
def _onehot_sc(types_hbm, out_hbm, types_v, buf):
    wid = lax.axis_index("s") * 2 + lax.axis_index("c")
    ones16 = jnp.ones((16,), jnp.float32)
    zeros16 = jnp.zeros((16,), jnp.float32)
    iota16 = lax.iota(jnp.int32, 16)

    def zero_body(i, carry):
        for r in range(N_TYPES):
            buf[r, pl.ds(i * 16, 16)] = zeros16
        return carry

    lax.fori_loop(0, GROUPS, zero_body, 0)

    base = wid * CHUNK
    pltpu.sync_copy(types_hbm.at[pl.ds(base, CHUNK)], types_v)
    for g in range(GROUPS):
        t = types_v[pl.ds(g * 16, 16)]
        plsc.store_scatter(buf, [t, iota16 + g * 16], ones16)
    pltpu.sync_copy(buf, out_hbm.at[:, pl.ds(base, CHUNK)])
    for g in range(GROUPS):
        t = types_v[pl.ds(g * 16, 16)]
        plsc.store_scatter(buf, [t, iota16 + g * 16], zeros16)

    base2 = NW * CHUNK + wid * CHUNK2
    pltpu.sync_copy(types_hbm.at[pl.ds(base2, CHUNK2)], types_v.at[pl.ds(0, CHUNK2)])
    for g in range(GROUPS2):
        t = types_v[pl.ds(g * 16, 16)]
        plsc.store_scatter(buf, [t, iota16 + g * 16], ones16)
    pltpu.sync_copy(
        buf.at[:, pl.ds(0, CHUNK2)], out_hbm.at[:, pl.ds(base2, CHUNK2)]
    )


def _onehot_block_tc(types_ref, o_ref):
    t = types_ref[:]
    rows = lax.broadcasted_iota(jnp.int32, (N_TYPES, t.shape[0]), 0)
    o_ref[...] = (rows == t[None, :]).astype(jnp.float32)


def _region_tc(sc_ref, types_ref, o_ref):
    del sc_ref
    _onehot_block_tc(types_ref, o_ref)


_region_call = pl.pallas_call(
    _region_tc,
    grid=(N_TC_BLOCKS,),
    in_specs=[
        pl.BlockSpec(memory_space=pl.ANY),
        pl.BlockSpec((TC_BLOCK,), lambda i: (i + TC_FIRST,)),
    ],
    out_specs=pl.BlockSpec((N_TYPES, TC_BLOCK), lambda i: (0, i + TC_FIRST)),
    out_shape=jax.ShapeDtypeStruct((N_TYPES, N_ATOMS), jnp.float32),
    input_output_aliases={0: 0},
)

_dup_call = pl.pallas_call(
    _onehot_block_tc,
    grid=(N_DUP_BLOCKS,),
    in_specs=[pl.BlockSpec((DUP_BLOCK,), lambda i: (i,))],
    out_specs=pl.BlockSpec((N_TYPES, DUP_BLOCK), lambda i: (0, i)),
    out_shape=jax.ShapeDtypeStruct((N_TYPES, N_ATOMS), jnp.float32),
)


def kernel(atom_types, pos):
    del pos
    types = atom_types.reshape(-1)
    sc_out = _onehot_sc(types)
    out1 = _region_call(sc_out, types)
    out2 = _dup_call(types)
    return (out1.T, out2.T)
